# baseline (speedup 1.0000x reference)
import functools

import jax
import jax.numpy as jnp
from jax import lax
from jax.experimental import pallas as pl
from jax.experimental.pallas import tpu as pltpu
from jax.experimental.pallas import tpu_sc as plsc

N = 1024 * 1024
C = 10
NC = 2
NS = 16
NW = NC * NS
PER_W = N // NW
CHUNK = 8192
NCHUNK = PER_W // CHUNK
G = CHUNK // 16

_mesh = plsc.VectorSubcoreMesh(core_axis_name="c", subcore_axis_name="s")


@functools.partial(
    pl.kernel,
    out_type=jax.ShapeDtypeStruct((N * C,), jnp.float32),
    mesh=_mesh,
    scratch_types=[
        pltpu.VMEM((CHUNK,), jnp.int32),
        pltpu.VMEM((CHUNK * C,), jnp.float32),
    ],
    compiler_params=pltpu.CompilerParams(needs_layout_passes=False),
)
def _onehot_sc(x_hbm, zeros_hbm, out_hbm, xin, outbuf):
    wid = lax.axis_index("s") * NC + lax.axis_index("c")
    base = wid * PER_W

    iota10 = lax.iota(jnp.int32, 16) * C
    ones = jnp.full((16,), 1.0, jnp.float32)
    zeros = jnp.zeros((16,), jnp.float32)

    pltpu.sync_copy(zeros_hbm, outbuf)

    for ci in range(NCHUNK):
        off = base + ci * CHUNK
        pltpu.sync_copy(x_hbm.at[pl.ds(off, CHUNK)], xin)

        @plsc.parallel_loop(0, G, 1, unroll=8, carry=iota10)
        def _set(j, basev):
            vals = xin[pl.ds(j * 16, 16)]
            plsc.store_scatter(outbuf, [basev + vals], ones)
            return basev + 16 * C

        pltpu.sync_copy(outbuf, out_hbm.at[pl.ds(off * C, CHUNK * C)])

        @plsc.parallel_loop(0, G, 1, unroll=8, carry=iota10)
        def _clr(j, basev):
            vals = xin[pl.ds(j * 16, 16)]
            plsc.store_scatter(outbuf, [basev + vals], zeros)
            return basev + 16 * C


def kernel(x):
    x_flat = x.reshape(N)
    zeros_stage = jnp.zeros((CHUNK * C,), jnp.float32)
    out_flat = _onehot_sc(x_flat, zeros_stage)
    return out_flat.reshape(1024, 1024, C)


# speedup vs baseline: 11.9189x; 11.9189x over previous
import functools

import jax
import jax.numpy as jnp
from jax import lax
from jax.experimental import pallas as pl
from jax.experimental.pallas import tpu as pltpu
from jax.experimental.pallas import tpu_sc as plsc

C = 10
NC = 2
NS = 16
NW = NC * NS
ROWS_W = 1024 // NW
RCHUNK = 8
NCHUNK = ROWS_W // RCHUNK
G = RCHUNK * 1024 // 16

_mesh = plsc.VectorSubcoreMesh(core_axis_name="c", subcore_axis_name="s")


@functools.partial(
    pl.kernel,
    out_type=jax.ShapeDtypeStruct((C, 1024, 1024), jnp.float32),
    mesh=_mesh,
    scratch_types=[
        pltpu.VMEM((RCHUNK, 1024), jnp.int32),
        pltpu.VMEM((C, RCHUNK, 1024), jnp.float32),
    ],
    compiler_params=pltpu.CompilerParams(needs_layout_passes=False),
)
def _onehot_sc(x_hbm, zeros_hbm, out_hbm, xin, outbuf):
    wid = lax.axis_index("s") * NC + lax.axis_index("c")
    base_row = wid * ROWS_W

    iota16 = lax.iota(jnp.int32, 16)
    ones = jnp.full((16,), 1.0, jnp.float32)
    zeros = jnp.zeros((16,), jnp.float32)

    pltpu.sync_copy(zeros_hbm, outbuf)

    for ci in range(NCHUNK):
        r0 = base_row + ci * RCHUNK
        pltpu.sync_copy(x_hbm.at[pl.ds(r0, RCHUNK), :], xin)

        for r in range(RCHUNK):
            row_vec = jnp.full((16,), r, jnp.int32)

            @plsc.parallel_loop(0, 1024 // 16, 1, unroll=8, carry=iota16)
            def _set(j, colv):
                vals = xin[r, pl.ds(j * 16, 16)]
                plsc.store_scatter(outbuf, [vals, row_vec, colv], ones)
                return colv + 16

        pltpu.sync_copy(outbuf, out_hbm.at[:, pl.ds(r0, RCHUNK), :])

        for r in range(RCHUNK):
            row_vec = jnp.full((16,), r, jnp.int32)

            @plsc.parallel_loop(0, 1024 // 16, 1, unroll=8, carry=iota16)
            def _clr(j, colv):
                vals = xin[r, pl.ds(j * 16, 16)]
                plsc.store_scatter(outbuf, [vals, row_vec, colv], zeros)
                return colv + 16


def kernel(x):
    x2d = x.reshape(1024, 1024)
    zeros_stage = jnp.zeros((C, RCHUNK, 1024), jnp.float32)
    out_cm = _onehot_sc(x2d, zeros_stage)
    return out_cm.transpose(1, 2, 0)
